# baseline (speedup 1.0000x reference)
import jax
import jax.numpy as jnp
from jax import lax
from jax.experimental import pallas as pl
from jax.experimental.pallas import tpu as pltpu

_T = 8
_G = 4
_NB = 8
_F = 64
_C = 512
_NF = _C // 4
_HW = 784
_BHW = 16
_NI = _HW // _BHW


def _wmat_emit(pooled_ref, cwm_ref, wcw0_ref, wcw1_ref, f1w_ref, f2w_ref,
               cb_ref, f1b_ref, f2b_ref, wcb_ref, w_ref):
    w_ref[...] = jnp.zeros_like(w_ref)

    def conv_t(mm, bias):
        a = mm[:, 0:1]
        b = mm[:, 1:2]
        c = mm[:, 2:3]
        z = jnp.zeros((1, 1), jnp.float32)
        return (b + jnp.concatenate([z, a[:-1]], axis=0)
                + jnp.concatenate([c[1:], z], axis=0) + bias)

    for n in range(_NB):
        pooled = pooled_ref[n * _T:(n + 1) * _T, :] * (1.0 / _HW)

        mm = jnp.dot(pooled, cwm_ref[...],
                     preferred_element_type=jnp.float32)
        oc = conv_t(mm, cb_ref[0, 0])
        h1 = jnp.maximum(
            jnp.dot(f1w_ref[...], oc, preferred_element_type=jnp.float32)
            + f1b_ref[...], 0.0)
        o2 = (jnp.dot(f2w_ref[...], h1, preferred_element_type=jnp.float32)
              + f2b_ref[...])
        offv = 4.0 * (jax.nn.sigmoid(o2) - 0.5)

        wm0 = jnp.dot(pooled, wcw0_ref[...],
                      preferred_element_type=jnp.float32)
        wm1 = jnp.dot(pooled, wcw1_ref[...],
                      preferred_element_type=jnp.float32)
        xw0 = 2.0 * jax.nn.sigmoid(conv_t(wm0, wcb_ref[0, 0]))
        xw1 = 2.0 * jax.nn.sigmoid(conv_t(wm1, wcb_ref[1, 0]))

        iota_t = lax.broadcasted_iota(jnp.int32, (_T, 1), 0)
        for g in range(_G):
            off_g = offv[g % 2, 0]
            if g >= 2:
                off_g = -off_g
            o0f = jnp.floor(off_g)
            o0 = o0f.astype(jnp.int32)
            frac = off_g - o0f
            w0 = 1.0 - frac
            w1 = frac
            xw = xw0 if (g % 2 == 0) else xw1
            s0 = iota_t + o0
            s1 = s0 + 1
            v0 = jnp.where((s0 >= 0) & (s0 < _T), 1.0, 0.0)
            v1 = jnp.where((s1 >= 0) & (s1 < _T), 1.0, 0.0)
            c0col = w0 * xw * v0
            c1col = w1 * xw * v1
            for d in range(-2, 3):
                m0 = jnp.where(o0 == d, 1.0, 0.0)
                m1 = jnp.where(o0 == d - 1, 1.0, 0.0)
                col = c0col * m0 + c1col * m1
                w_ref[d + 2, n * _T:(n + 1) * _T, g * 32:(g + 1) * 32] = (
                    jnp.broadcast_to(col, (_T, 32)))


def _mega_body(xp_ref, desc_ref, cwm_ref, wcw0_ref, wcw1_ref, f1w_ref,
               f2w_ref, cb_ref, f1b_ref, f2b_ref, wcb_ref, o_ref,
               dbuf_ref, acc_ref, w_ref):
    e = pl.program_id(0)
    i = pl.program_id(1)
    j = pl.program_id(2)

    @pl.when((e == 0) & (i == 0) & (j == 0))
    def _init():
        acc_ref[...] = jnp.zeros_like(acc_ref)

    @pl.when(e == 0)
    def _passthrough():
        o_ref[...] = xp_ref[...]

    @pl.when((e == 0) & (j == 0))
    def _stash():
        d = desc_ref[...]
        dbuf_ref[pl.ds(i * _BHW, _BHW)] = d
        acc_ref[...] += jnp.sum(d, axis=0)

    @pl.when((e == 0) & (i == _NI - 1) & (j == 2))
    def _emit():
        _wmat_emit(acc_ref, cwm_ref, wcw0_ref, wcw1_ref, f1w_ref, f2w_ref,
                   cb_ref, f1b_ref, f2b_ref, wcb_ref, w_ref)

    @pl.when(e == 1)
    def _blend():
        db = dbuf_ref[pl.ds(i * _BHW, _BHW)]
        acc = w_ref[2][None] * db
        for d in (-2, -1, 1, 2):
            if d < 0:
                shifted = jnp.concatenate(
                    [jnp.zeros((_BHW, -d, _NF), jnp.float32),
                     db[:, :_F + d, :]], axis=1)
            else:
                shifted = jnp.concatenate(
                    [db[:, d:, :],
                     jnp.zeros((_BHW, d, _NF), jnp.float32)], axis=1)
            acc += w_ref[d + 2][None] * shifted
        o_ref[...] = acc


def kernel(x, off_conv_w, off_conv_b, off_fc1_w, off_fc1_b, off_fc2_w,
           off_fc2_b, w_conv_w, w_conv_b):
    n, c, h, w = x.shape
    hw = h * w
    xv = jnp.transpose(x, (2, 3, 0, 1)).reshape(hw, _F, _C)

    cwm = off_conv_w[0]
    wcw0 = w_conv_w[0]
    wcw1 = w_conv_w[1]
    cb = off_conv_b.reshape(1, 1)
    f1b = off_fc1_b.reshape(_T, 1)
    f2b = off_fc2_b.reshape(2, 1)
    wcb = w_conv_b.reshape(2, 1)

    def xp_idx(e, i, j):
        return (jnp.where(e == 0, i, _NI - 1), 0, jnp.where(e == 0, j + 1, 3))

    def desc_idx(e, i, j):
        return (jnp.where(e == 0, i, _NI - 1), 0, 0)

    def o_idx(e, i, j):
        return (i, 0, jnp.where(e == 0, j + 1, 0))

    small = lambda shape: pl.BlockSpec(shape, lambda e, i, j: (0,) * len(shape))
    outv = pl.pallas_call(
        _mega_body,
        grid=(2, _NI, 3),
        in_specs=[
            pl.BlockSpec((_BHW, _F, _NF), xp_idx),
            pl.BlockSpec((_BHW, _F, _NF), desc_idx),
            small((_NF, 3)), small((_NF, 3)), small((_NF, 3)),
            small((_T, _T)), small((2, _T)),
            small((1, 1)), small((_T, 1)), small((2, 1)), small((2, 1)),
        ],
        out_specs=pl.BlockSpec((_BHW, _F, _NF), o_idx),
        out_shape=jax.ShapeDtypeStruct((hw, _F, _C), jnp.float32),
        scratch_shapes=[
            pltpu.VMEM((_HW, _F, _NF), jnp.float32),
            pltpu.VMEM((_F, _NF), jnp.float32),
            pltpu.VMEM((5, _F, _NF), jnp.float32),
        ],
        compiler_params=pltpu.CompilerParams(
            dimension_semantics=("arbitrary", "arbitrary", "arbitrary")),
    )(xv, xv, cwm, wcw0, wcw1, off_fc1_w, off_fc2_w, cb, f1b, f2b, wcb)

    return outv.reshape(h, w, n, c).transpose(2, 3, 0, 1)


# speedup vs baseline: 2.1042x; 2.1042x over previous
import jax
import jax.numpy as jnp
from jax import lax
from jax.experimental import pallas as pl
from jax.experimental.pallas import tpu as pltpu

_T = 8
_G = 4
_NB = 8
_F = 64
_C = 512
_NF = _C // 4
_HW = 784
_BHW = 28


def _wmat_body(x_ref, cwm_ref, wcw0_ref, wcw1_ref, f1w_ref, f2w_ref,
               cb_ref, f1b_ref, f2b_ref, wcb_ref, w_ref, acc_ref):
    i = pl.program_id(0)

    @pl.when(i == 0)
    def _init():
        acc_ref[...] = jnp.zeros_like(acc_ref)

    acc_ref[...] += jnp.sum(x_ref[...], axis=0)

    @pl.when(i == pl.num_programs(0) - 1)
    def _emit():
        _wmat_emit(acc_ref, cwm_ref, wcw0_ref, wcw1_ref, f1w_ref, f2w_ref,
                   cb_ref, f1b_ref, f2b_ref, wcb_ref, w_ref)


def _wmat_emit(pooled_ref, cwm_ref, wcw0_ref, wcw1_ref, f1w_ref, f2w_ref,
               cb_ref, f1b_ref, f2b_ref, wcb_ref, w_ref):
    w_ref[...] = jnp.zeros_like(w_ref)

    def conv_t(mm, bias):
        a = mm[:, 0:1]
        b = mm[:, 1:2]
        c = mm[:, 2:3]
        z = jnp.zeros((1, 1), jnp.float32)
        return (b + jnp.concatenate([z, a[:-1]], axis=0)
                + jnp.concatenate([c[1:], z], axis=0) + bias)

    for n in range(_NB):
        pooled = pooled_ref[n * _T:(n + 1) * _T, :] * (1.0 / _HW)

        mm = jnp.dot(pooled, cwm_ref[...],
                     preferred_element_type=jnp.float32)
        oc = conv_t(mm, cb_ref[0, 0])
        h1 = jnp.maximum(
            jnp.dot(f1w_ref[...], oc, preferred_element_type=jnp.float32)
            + f1b_ref[...], 0.0)
        o2 = (jnp.dot(f2w_ref[...], h1, preferred_element_type=jnp.float32)
              + f2b_ref[...])
        offv = 4.0 * (jax.nn.sigmoid(o2) - 0.5)

        wm0 = jnp.dot(pooled, wcw0_ref[...],
                      preferred_element_type=jnp.float32)
        wm1 = jnp.dot(pooled, wcw1_ref[...],
                      preferred_element_type=jnp.float32)
        xw0 = 2.0 * jax.nn.sigmoid(conv_t(wm0, wcb_ref[0, 0]))
        xw1 = 2.0 * jax.nn.sigmoid(conv_t(wm1, wcb_ref[1, 0]))

        iota_t = lax.broadcasted_iota(jnp.int32, (_T, 1), 0)
        for g in range(_G):
            off_g = offv[g % 2, 0]
            if g >= 2:
                off_g = -off_g
            o0f = jnp.floor(off_g)
            o0 = o0f.astype(jnp.int32)
            frac = off_g - o0f
            w0 = 1.0 - frac
            w1 = frac
            xw = xw0 if (g % 2 == 0) else xw1
            s0 = iota_t + o0
            s1 = s0 + 1
            v0 = jnp.where((s0 >= 0) & (s0 < _T), 1.0, 0.0)
            v1 = jnp.where((s1 >= 0) & (s1 < _T), 1.0, 0.0)
            c0col = w0 * xw * v0
            c1col = w1 * xw * v1
            for d in range(-2, 3):
                m0 = jnp.where(o0 == d, 1.0, 0.0)
                m1 = jnp.where(o0 == d - 1, 1.0, 0.0)
                col = c0col * m0 + c1col * m1
                w_ref[d + 2, n * _T:(n + 1) * _T, g * 32:(g + 1) * 32] = (
                    jnp.broadcast_to(col, (_T, 32)))


def _data_body(x_ref, w_ref, o_ref):
    xb = x_ref[...]
    desc = xb[:, :, :_NF]
    acc = w_ref[2][None] * desc
    for d in (-2, -1, 1, 2):
        if d < 0:
            shifted = jnp.concatenate(
                [jnp.zeros((_BHW, -d, _NF), jnp.float32),
                 desc[:, :_F + d, :]], axis=1)
        else:
            shifted = jnp.concatenate(
                [desc[:, d:, :],
                 jnp.zeros((_BHW, d, _NF), jnp.float32)], axis=1)
        acc += w_ref[d + 2][None] * shifted
    o_ref[:, :, :_NF] = acc
    o_ref[:, :, _NF:] = xb[:, :, _NF:]


def kernel(x, off_conv_w, off_conv_b, off_fc1_w, off_fc1_b, off_fc2_w,
           off_fc2_b, w_conv_w, w_conv_b):
    n, c, h, w = x.shape
    hw = h * w
    xv = jnp.transpose(x, (2, 3, 0, 1)).reshape(hw, _F, _C)

    cwm = off_conv_w[0]
    wcw0 = w_conv_w[0]
    wcw1 = w_conv_w[1]
    cb = off_conv_b.reshape(1, 1)
    f1b = off_fc1_b.reshape(_T, 1)
    f2b = off_fc2_b.reshape(2, 1)
    wcb = w_conv_b.reshape(2, 1)

    small = lambda shape: pl.BlockSpec(shape, lambda i: (0,) * len(shape))
    wmat = pl.pallas_call(
        _wmat_body,
        grid=(hw // _BHW,),
        in_specs=[
            pl.BlockSpec((_BHW, _F, _NF), lambda i: (i, 0, 0)),
            small((_NF, 3)), small((_NF, 3)), small((_NF, 3)),
            small((_T, _T)), small((2, _T)),
            small((1, 1)), small((_T, 1)), small((2, 1)), small((2, 1)),
        ],
        out_specs=pl.BlockSpec((5, _F, _NF), lambda i: (0, 0, 0)),
        out_shape=jax.ShapeDtypeStruct((5, _F, _NF), jnp.float32),
        scratch_shapes=[pltpu.VMEM((_F, _NF), jnp.float32)],
        compiler_params=pltpu.CompilerParams(
            dimension_semantics=("arbitrary",)),
    )(xv, cwm, wcw0, wcw1, off_fc1_w, off_fc2_w, cb, f1b, f2b, wcb)

    outv = pl.pallas_call(
        _data_body,
        grid=(hw // _BHW,),
        in_specs=[
            pl.BlockSpec((_BHW, _F, _C), lambda i: (i, 0, 0)),
            pl.BlockSpec((5, _F, _NF), lambda i: (0, 0, 0)),
        ],
        out_specs=pl.BlockSpec((_BHW, _F, _C), lambda i: (i, 0, 0)),
        out_shape=jax.ShapeDtypeStruct((hw, _F, _C), jnp.float32),
        compiler_params=pltpu.CompilerParams(
            dimension_semantics=("arbitrary",)),
    )(xv, wmat)

    return outv.reshape(h, w, n, c).transpose(2, 3, 0, 1)


# speedup vs baseline: 2.2738x; 1.0806x over previous
import jax
import jax.numpy as jnp
from jax import lax
from jax.experimental import pallas as pl
from jax.experimental.pallas import tpu as pltpu

_T = 8
_G = 4
_NB = 8
_F = 64
_C = 512
_NF = _C // 4
_HW = 784
_BHW = 49


def _wmat_body(x_ref, cwm_ref, wcw0_ref, wcw1_ref, f1w_ref, f2w_ref,
               cb_ref, f1b_ref, f2b_ref, wcb_ref, w_ref, acc_ref):
    i = pl.program_id(0)

    @pl.when(i == 0)
    def _init():
        acc_ref[...] = jnp.zeros_like(acc_ref)

    acc_ref[...] += jnp.sum(x_ref[...], axis=0)

    @pl.when(i == pl.num_programs(0) - 1)
    def _emit():
        _wmat_emit(acc_ref, cwm_ref, wcw0_ref, wcw1_ref, f1w_ref, f2w_ref,
                   cb_ref, f1b_ref, f2b_ref, wcb_ref, w_ref)


def _wmat_emit(pooled_ref, cwm_ref, wcw0_ref, wcw1_ref, f1w_ref, f2w_ref,
               cb_ref, f1b_ref, f2b_ref, wcb_ref, w_ref):
    w_ref[...] = jnp.zeros_like(w_ref)

    def conv_t(mm, bias):
        a = mm[:, 0:1]
        b = mm[:, 1:2]
        c = mm[:, 2:3]
        z = jnp.zeros((1, 1), jnp.float32)
        return (b + jnp.concatenate([z, a[:-1]], axis=0)
                + jnp.concatenate([c[1:], z], axis=0) + bias)

    for n in range(_NB):
        pooled = pooled_ref[n * _T:(n + 1) * _T, :] * (1.0 / _HW)

        mm = jnp.dot(pooled, cwm_ref[...],
                     preferred_element_type=jnp.float32)
        oc = conv_t(mm, cb_ref[0, 0])
        h1 = jnp.maximum(
            jnp.dot(f1w_ref[...], oc, preferred_element_type=jnp.float32)
            + f1b_ref[...], 0.0)
        o2 = (jnp.dot(f2w_ref[...], h1, preferred_element_type=jnp.float32)
              + f2b_ref[...])
        offv = 4.0 * (jax.nn.sigmoid(o2) - 0.5)

        wm0 = jnp.dot(pooled, wcw0_ref[...],
                      preferred_element_type=jnp.float32)
        wm1 = jnp.dot(pooled, wcw1_ref[...],
                      preferred_element_type=jnp.float32)
        xw0 = 2.0 * jax.nn.sigmoid(conv_t(wm0, wcb_ref[0, 0]))
        xw1 = 2.0 * jax.nn.sigmoid(conv_t(wm1, wcb_ref[1, 0]))

        iota_t = lax.broadcasted_iota(jnp.int32, (_T, 1), 0)
        for g in range(_G):
            off_g = offv[g % 2, 0]
            if g >= 2:
                off_g = -off_g
            o0f = jnp.floor(off_g)
            o0 = o0f.astype(jnp.int32)
            frac = off_g - o0f
            w0 = 1.0 - frac
            w1 = frac
            xw = xw0 if (g % 2 == 0) else xw1
            s0 = iota_t + o0
            s1 = s0 + 1
            v0 = jnp.where((s0 >= 0) & (s0 < _T), 1.0, 0.0)
            v1 = jnp.where((s1 >= 0) & (s1 < _T), 1.0, 0.0)
            c0col = w0 * xw * v0
            c1col = w1 * xw * v1
            for d in range(-2, 3):
                m0 = jnp.where(o0 == d, 1.0, 0.0)
                m1 = jnp.where(o0 == d - 1, 1.0, 0.0)
                col = c0col * m0 + c1col * m1
                w_ref[d + 2, n * _T:(n + 1) * _T, g * 32:(g + 1) * 32] = (
                    jnp.broadcast_to(col, (_T, 32)))


def _data_body(x_ref, w_ref, o_ref):
    xb = x_ref[...]
    desc = xb[:, :, :_NF]
    acc = w_ref[2][None] * desc
    for d in (-2, -1, 1, 2):
        if d < 0:
            shifted = jnp.concatenate(
                [jnp.zeros((_BHW, -d, _NF), jnp.float32),
                 desc[:, :_F + d, :]], axis=1)
        else:
            shifted = jnp.concatenate(
                [desc[:, d:, :],
                 jnp.zeros((_BHW, d, _NF), jnp.float32)], axis=1)
        acc += w_ref[d + 2][None] * shifted
    o_ref[:, :, :_NF] = acc
    o_ref[:, :, _NF:] = xb[:, :, _NF:]


def kernel(x, off_conv_w, off_conv_b, off_fc1_w, off_fc1_b, off_fc2_w,
           off_fc2_b, w_conv_w, w_conv_b):
    n, c, h, w = x.shape
    hw = h * w
    xv = jnp.transpose(x, (2, 3, 0, 1)).reshape(hw, _F, _C)

    cwm = off_conv_w[0]
    wcw0 = w_conv_w[0]
    wcw1 = w_conv_w[1]
    cb = off_conv_b.reshape(1, 1)
    f1b = off_fc1_b.reshape(_T, 1)
    f2b = off_fc2_b.reshape(2, 1)
    wcb = w_conv_b.reshape(2, 1)

    small = lambda shape: pl.BlockSpec(shape, lambda i: (0,) * len(shape))
    wmat = pl.pallas_call(
        _wmat_body,
        grid=(hw // _BHW,),
        in_specs=[
            pl.BlockSpec((_BHW, _F, _NF), lambda i: (i, 0, 0)),
            small((_NF, 3)), small((_NF, 3)), small((_NF, 3)),
            small((_T, _T)), small((2, _T)),
            small((1, 1)), small((_T, 1)), small((2, 1)), small((2, 1)),
        ],
        out_specs=pl.BlockSpec((5, _F, _NF), lambda i: (0, 0, 0)),
        out_shape=jax.ShapeDtypeStruct((5, _F, _NF), jnp.float32),
        scratch_shapes=[pltpu.VMEM((_F, _NF), jnp.float32)],
        compiler_params=pltpu.CompilerParams(
            dimension_semantics=("arbitrary",)),
    )(xv, cwm, wcw0, wcw1, off_fc1_w, off_fc2_w, cb, f1b, f2b, wcb)

    outv = pl.pallas_call(
        _data_body,
        grid=(hw // _BHW,),
        in_specs=[
            pl.BlockSpec((_BHW, _F, _C), lambda i: (i, 0, 0)),
            pl.BlockSpec((5, _F, _NF), lambda i: (0, 0, 0)),
        ],
        out_specs=pl.BlockSpec((_BHW, _F, _C), lambda i: (i, 0, 0)),
        out_shape=jax.ShapeDtypeStruct((hw, _F, _C), jnp.float32),
        compiler_params=pltpu.CompilerParams(
            dimension_semantics=("arbitrary",)),
    )(xv, wmat)

    return outv.reshape(h, w, n, c).transpose(2, 3, 0, 1)


# speedup vs baseline: 2.3010x; 1.0120x over previous
import jax
import jax.numpy as jnp
from jax import lax
from jax.experimental import pallas as pl
from jax.experimental.pallas import tpu as pltpu

_T = 8
_G = 4
_NB = 8
_F = 64
_C = 512
_NF = _C // 4
_HW = 784
_BHW = 56


def _wmat_body(x_ref, cwm_ref, wcw0_ref, wcw1_ref, f1w_ref, f2w_ref,
               cb_ref, f1b_ref, f2b_ref, wcb_ref, w_ref, acc_ref):
    i = pl.program_id(0)

    @pl.when(i == 0)
    def _init():
        acc_ref[...] = jnp.zeros_like(acc_ref)

    acc_ref[...] += jnp.sum(x_ref[...], axis=0)

    @pl.when(i == pl.num_programs(0) - 1)
    def _emit():
        _wmat_emit(acc_ref, cwm_ref, wcw0_ref, wcw1_ref, f1w_ref, f2w_ref,
                   cb_ref, f1b_ref, f2b_ref, wcb_ref, w_ref)


def _wmat_emit(pooled_ref, cwm_ref, wcw0_ref, wcw1_ref, f1w_ref, f2w_ref,
               cb_ref, f1b_ref, f2b_ref, wcb_ref, w_ref):
    w_ref[...] = jnp.zeros_like(w_ref)

    def conv_t(mm, bias):
        a = mm[:, 0:1]
        b = mm[:, 1:2]
        c = mm[:, 2:3]
        z = jnp.zeros((1, 1), jnp.float32)
        return (b + jnp.concatenate([z, a[:-1]], axis=0)
                + jnp.concatenate([c[1:], z], axis=0) + bias)

    for n in range(_NB):
        pooled = pooled_ref[n * _T:(n + 1) * _T, :] * (1.0 / _HW)

        mm = jnp.dot(pooled, cwm_ref[...],
                     preferred_element_type=jnp.float32)
        oc = conv_t(mm, cb_ref[0, 0])
        h1 = jnp.maximum(
            jnp.dot(f1w_ref[...], oc, preferred_element_type=jnp.float32)
            + f1b_ref[...], 0.0)
        o2 = (jnp.dot(f2w_ref[...], h1, preferred_element_type=jnp.float32)
              + f2b_ref[...])
        offv = 4.0 * (jax.nn.sigmoid(o2) - 0.5)

        wm0 = jnp.dot(pooled, wcw0_ref[...],
                      preferred_element_type=jnp.float32)
        wm1 = jnp.dot(pooled, wcw1_ref[...],
                      preferred_element_type=jnp.float32)
        xw0 = 2.0 * jax.nn.sigmoid(conv_t(wm0, wcb_ref[0, 0]))
        xw1 = 2.0 * jax.nn.sigmoid(conv_t(wm1, wcb_ref[1, 0]))

        iota_t = lax.broadcasted_iota(jnp.int32, (_T, 1), 0)
        for g in range(_G):
            off_g = offv[g % 2, 0]
            if g >= 2:
                off_g = -off_g
            o0f = jnp.floor(off_g)
            o0 = o0f.astype(jnp.int32)
            frac = off_g - o0f
            w0 = 1.0 - frac
            w1 = frac
            xw = xw0 if (g % 2 == 0) else xw1
            s0 = iota_t + o0
            s1 = s0 + 1
            v0 = jnp.where((s0 >= 0) & (s0 < _T), 1.0, 0.0)
            v1 = jnp.where((s1 >= 0) & (s1 < _T), 1.0, 0.0)
            c0col = w0 * xw * v0
            c1col = w1 * xw * v1
            for d in range(-2, 3):
                m0 = jnp.where(o0 == d, 1.0, 0.0)
                m1 = jnp.where(o0 == d - 1, 1.0, 0.0)
                col = c0col * m0 + c1col * m1
                w_ref[d + 2, n * _T:(n + 1) * _T, g * 32:(g + 1) * 32] = (
                    jnp.broadcast_to(col, (_T, 32)))


def _data_body(x_ref, w_ref, o_ref):
    xb = x_ref[...]
    desc = xb[:, :, :_NF]
    acc = w_ref[2][None] * desc
    for d in (-2, -1, 1, 2):
        if d < 0:
            shifted = jnp.concatenate(
                [jnp.zeros((_BHW, -d, _NF), jnp.float32),
                 desc[:, :_F + d, :]], axis=1)
        else:
            shifted = jnp.concatenate(
                [desc[:, d:, :],
                 jnp.zeros((_BHW, d, _NF), jnp.float32)], axis=1)
        acc += w_ref[d + 2][None] * shifted
    o_ref[:, :, :_NF] = acc
    o_ref[:, :, _NF:] = xb[:, :, _NF:]


def kernel(x, off_conv_w, off_conv_b, off_fc1_w, off_fc1_b, off_fc2_w,
           off_fc2_b, w_conv_w, w_conv_b):
    n, c, h, w = x.shape
    hw = h * w
    xv = jnp.transpose(x, (2, 3, 0, 1)).reshape(hw, _F, _C)

    cwm = off_conv_w[0]
    wcw0 = w_conv_w[0]
    wcw1 = w_conv_w[1]
    cb = off_conv_b.reshape(1, 1)
    f1b = off_fc1_b.reshape(_T, 1)
    f2b = off_fc2_b.reshape(2, 1)
    wcb = w_conv_b.reshape(2, 1)

    small = lambda shape: pl.BlockSpec(shape, lambda i: (0,) * len(shape))
    wmat = pl.pallas_call(
        _wmat_body,
        grid=(hw // _BHW,),
        in_specs=[
            pl.BlockSpec((_BHW, _F, _NF), lambda i: (i, 0, 0)),
            small((_NF, 3)), small((_NF, 3)), small((_NF, 3)),
            small((_T, _T)), small((2, _T)),
            small((1, 1)), small((_T, 1)), small((2, 1)), small((2, 1)),
        ],
        out_specs=pl.BlockSpec((5, _F, _NF), lambda i: (0, 0, 0)),
        out_shape=jax.ShapeDtypeStruct((5, _F, _NF), jnp.float32),
        scratch_shapes=[pltpu.VMEM((_F, _NF), jnp.float32)],
        compiler_params=pltpu.CompilerParams(
            dimension_semantics=("arbitrary",)),
    )(xv, cwm, wcw0, wcw1, off_fc1_w, off_fc2_w, cb, f1b, f2b, wcb)

    outv = pl.pallas_call(
        _data_body,
        grid=(hw // _BHW,),
        in_specs=[
            pl.BlockSpec((_BHW, _F, _C), lambda i: (i, 0, 0)),
            pl.BlockSpec((5, _F, _NF), lambda i: (0, 0, 0)),
        ],
        out_specs=pl.BlockSpec((_BHW, _F, _C), lambda i: (i, 0, 0)),
        out_shape=jax.ShapeDtypeStruct((hw, _F, _C), jnp.float32),
        compiler_params=pltpu.CompilerParams(
            dimension_semantics=("arbitrary",)),
    )(xv, wmat)

    return outv.reshape(h, w, n, c).transpose(2, 3, 0, 1)
